# baseline (speedup 1.0000x reference)
import functools

import jax
import jax.numpy as jnp
from jax import lax
from jax.experimental import pallas as pl
from jax.experimental.pallas import tpu as pltpu
from jax.experimental.pallas import tpu_sc as plsc

NC = 2
NS = 16
NW = NC * NS
CHUNK = 128
OUT_SIZE = 64


def _sc_edge_scatter(st_t, src3, dst3, w3, zeros, n_nodes):
    b = st_t.shape[1]
    n_chunks = src3.shape[1]
    zrows = zeros.shape[0]
    nslices = n_nodes // zrows
    mesh = plsc.VectorSubcoreMesh(core_axis_name="c", subcore_axis_name="s")

    @functools.partial(
        pl.kernel,
        out_type=jax.ShapeDtypeStruct((NC, n_nodes, b), jnp.float32),
        mesh=mesh,
        scratch_types=[
            pltpu.VMEM((n_chunks, CHUNK), jnp.int32),
            pltpu.VMEM((n_chunks, CHUNK), jnp.int32),
            pltpu.VMEM((n_chunks, CHUNK), jnp.float32),
            pltpu.VMEM((CHUNK, b), jnp.float32),
            pltpu.VMEM((CHUNK, b), jnp.float32),
            pltpu.VMEM_SHARED((n_nodes, b), jnp.float32),
            pltpu.SemaphoreType.DMA,
            pltpu.SemaphoreType.DMA,
            pltpu.SemaphoreType.DMA,
            pltpu.SemaphoreType.DMA,
        ],
        compiler_params=pltpu.CompilerParams(use_tc_tiling_on_sc=False),
    )
    def sc_kern(st_hbm, src_hbm, dst_hbm, w_hbm, zeros_hbm, out_hbm,
                src_v, dst_v, w_v, rows_a, rows_b, acc_sh,
                gsem_a, gsem_b, ssem_a, ssem_b):
        cid = lax.axis_index("c")
        sid = lax.axis_index("s")
        wid = sid * NC + cid

        pltpu.sync_copy(src_hbm.at[wid], src_v)
        pltpu.sync_copy(dst_hbm.at[wid], dst_v)
        pltpu.sync_copy(w_hbm.at[wid], w_v)

        @pl.when(sid < nslices)
        def _zero():
            pltpu.sync_copy(zeros_hbm,
                            acc_sh.at[pl.ds(sid * zrows, zrows)])
        plsc.subcore_barrier()

        def gather_start(c, buf, sem):
            return

        def gather_wait(buf, sem):
            return

        def scale(c, buf):
            def group_body(g, _):
                wvec = w_v[c, pl.ds(g * 16, 16)]
                for l in range(16):
                    wb = jnp.broadcast_to(wvec[l], (16,))
                    i = g * 16 + l
                    for j in range(b // 16):
                        sl = pl.ds(j * 16, 16)
                        buf[i, sl] = buf[i, sl] * wb
                return 0
            lax.fori_loop(0, CHUNK // 16, group_body, 0)

        def scatter_start(c, buf, sem):
            return

        def scatter_wait(c, buf, sem):
            return

        gather_start(0, rows_a, gsem_a)

        def pair_body(k, _):
            c0 = 2 * k
            gather_wait(rows_a, gsem_a)
            gather_start(c0 + 1, rows_b, gsem_b)
            scale(c0, rows_a)
            scatter_start(c0, rows_a, ssem_a)
            gather_wait(rows_b, gsem_b)
            scale(c0 + 1, rows_b)
            scatter_start(c0 + 1, rows_b, ssem_b)
            scatter_wait(c0, rows_a, ssem_a)
            gather_start(c0 + 2, rows_a, gsem_a)
            scatter_wait(c0 + 1, rows_b, ssem_b)
            return 0
        lax.fori_loop(0, (n_chunks - 2) // 2, pair_body, 0)

        c1 = n_chunks - 2
        gather_wait(rows_a, gsem_a)
        gather_start(c1 + 1, rows_b, gsem_b)
        scale(c1, rows_a)
        scatter_start(c1, rows_a, ssem_a)
        gather_wait(rows_b, gsem_b)
        scale(c1 + 1, rows_b)
        scatter_start(c1 + 1, rows_b, ssem_b)
        scatter_wait(c1, rows_a, ssem_a)
        scatter_wait(c1 + 1, rows_b, ssem_b)

        plsc.subcore_barrier()

        @pl.when(sid < nslices)
        def _writeback():
            pltpu.sync_copy(acc_sh.at[pl.ds(sid * zrows, zrows)],
                            out_hbm.at[cid, pl.ds(sid * zrows, zrows)])

    return sc_kern(st_t, src3, dst3, w3, zeros)


def _tc_finish(partials, bias2, resp2):
    n, b = partials.shape[1], partials.shape[2]

    def tc_kern(p_ref, b_ref, r_ref, o_ref):
        agg = p_ref[0] + p_ref[1]
        o_ref[...] = jnp.tanh(b_ref[...] + r_ref[...] * agg)

    return pl.pallas_call(
        tc_kern,
        out_shape=jax.ShapeDtypeStruct((n, b), jnp.float32),
    )(partials, bias2, resp2)


@jax.jit
def kernel(x, states, edge_index, edge_weight, node_bias, node_response):
    n_nodes = node_bias.shape[0]
    n_edges = edge_weight.shape[0]
    in_size = x.shape[1]
    st = states.at[:, :in_size].set(x)
    st_t = st.T

    grain = NW * CHUNK * 2
    n_pad = (-n_edges) % grain
    pad_idx = jnp.arange(n_pad, dtype=jnp.int32) % n_nodes
    src = jnp.concatenate([edge_index[0], pad_idx])
    dst = jnp.concatenate([edge_index[1], pad_idx])
    w = jnp.pad(edge_weight, (0, n_pad))
    n_chunks = (n_edges + n_pad) // (NW * CHUNK)
    src3 = src.reshape(NW, n_chunks, CHUNK)
    dst3 = dst.reshape(NW, n_chunks, CHUNK)
    w3 = w.reshape(NW, n_chunks, CHUNK)

    zeros = jnp.zeros((1000, st_t.shape[1]), jnp.float32)
    partials = _sc_edge_scatter(st_t, src3, dst3, w3, zeros, n_nodes)
    act = _tc_finish(partials,
                     node_bias.reshape(n_nodes, 1),
                     node_response.reshape(n_nodes, 1))
    new_states = act.T
    new_states = new_states.at[:, :in_size].set(x)
    y = new_states[:, -OUT_SIZE:]
    return (y, new_states)


# speedup vs baseline: 1.3766x; 1.3766x over previous
import functools

import jax
import jax.numpy as jnp
from jax import lax
from jax.experimental import pallas as pl
from jax.experimental.pallas import tpu as pltpu
from jax.experimental.pallas import tpu_sc as plsc

NC = 2
NS = 16
NW = NC * NS
CHUNK = 128
OUT_SIZE = 64


def _sc_edge_scatter(st_t, src3, dst3, w3, zeros, n_nodes):
    b = st_t.shape[1]
    n_chunks = src3.shape[1]
    zrows = zeros.shape[0]
    nslices = n_nodes // zrows
    mesh = plsc.VectorSubcoreMesh(core_axis_name="c", subcore_axis_name="s")

    @functools.partial(
        pl.kernel,
        out_type=jax.ShapeDtypeStruct((NC, n_nodes, b), jnp.float32),
        mesh=mesh,
        scratch_types=[
            pltpu.VMEM((n_chunks, CHUNK), jnp.int32),
            pltpu.VMEM((n_chunks, CHUNK), jnp.int32),
            pltpu.VMEM((n_chunks, CHUNK), jnp.float32),
            pltpu.VMEM((CHUNK, b), jnp.float32),
            pltpu.VMEM((CHUNK, b), jnp.float32),
            pltpu.VMEM_SHARED((n_nodes, b), jnp.float32),
            pltpu.SemaphoreType.DMA,
            pltpu.SemaphoreType.DMA,
            pltpu.SemaphoreType.DMA,
            pltpu.SemaphoreType.DMA,
        ],
        compiler_params=pltpu.CompilerParams(use_tc_tiling_on_sc=False),
    )
    def sc_kern(st_hbm, src_hbm, dst_hbm, w_hbm, zeros_hbm, out_hbm,
                src_v, dst_v, w_v, rows_a, rows_b, acc_sh,
                gsem_a, gsem_b, ssem_a, ssem_b):
        cid = lax.axis_index("c")
        sid = lax.axis_index("s")
        wid = sid * NC + cid

        pltpu.sync_copy(src_hbm.at[wid], src_v)
        pltpu.sync_copy(dst_hbm.at[wid], dst_v)
        pltpu.sync_copy(w_hbm.at[wid], w_v)

        @pl.when(sid < nslices)
        def _zero():
            pltpu.sync_copy(zeros_hbm,
                            acc_sh.at[pl.ds(sid * zrows, zrows)])
        plsc.subcore_barrier()

        def gather_start(c, buf, sem):
            pltpu.async_copy(st_hbm.at[src_v.at[c]], buf, sem)

        def gather_wait(buf, sem):
            pltpu.make_async_copy(st_hbm.at[src_v.at[0]], buf, sem).wait()

        def scale(c, buf):
            for g in range(CHUNK // 16):
                wvec = w_v[c, pl.ds(g * 16, 16)]
                for l in range(16):
                    wb = jnp.broadcast_to(wvec[l], (16,))
                    i = g * 16 + l
                    for j in range(b // 16):
                        sl = pl.ds(j * 16, 16)
                        buf[i, sl] = buf[i, sl] * wb

        def scatter_start(c, buf, sem):
            pltpu.async_copy(buf, acc_sh.at[dst_v.at[c]], sem, add=True)

        def scatter_wait(c, buf, sem):
            pltpu.make_async_copy(buf, acc_sh.at[dst_v.at[c]], sem).wait()

        gather_start(0, rows_a, gsem_a)

        def pair_body(k, _):
            c0 = 2 * k
            gather_wait(rows_a, gsem_a)
            gather_start(c0 + 1, rows_b, gsem_b)
            scale(c0, rows_a)
            scatter_start(c0, rows_a, ssem_a)
            gather_wait(rows_b, gsem_b)
            scale(c0 + 1, rows_b)
            scatter_start(c0 + 1, rows_b, ssem_b)
            scatter_wait(c0, rows_a, ssem_a)
            gather_start(c0 + 2, rows_a, gsem_a)
            scatter_wait(c0 + 1, rows_b, ssem_b)
            return 0
        lax.fori_loop(0, (n_chunks - 2) // 2, pair_body, 0)

        c1 = n_chunks - 2
        gather_wait(rows_a, gsem_a)
        gather_start(c1 + 1, rows_b, gsem_b)
        scale(c1, rows_a)
        scatter_start(c1, rows_a, ssem_a)
        gather_wait(rows_b, gsem_b)
        scale(c1 + 1, rows_b)
        scatter_start(c1 + 1, rows_b, ssem_b)
        scatter_wait(c1, rows_a, ssem_a)
        scatter_wait(c1 + 1, rows_b, ssem_b)

        plsc.subcore_barrier()

        @pl.when(sid < nslices)
        def _writeback():
            pltpu.sync_copy(acc_sh.at[pl.ds(sid * zrows, zrows)],
                            out_hbm.at[cid, pl.ds(sid * zrows, zrows)])

    return sc_kern(st_t, src3, dst3, w3, zeros)


def _tc_finish(partials, bias2, resp2):
    n, b = partials.shape[1], partials.shape[2]

    def tc_kern(p_ref, b_ref, r_ref, o_ref):
        agg = p_ref[0] + p_ref[1]
        o_ref[...] = jnp.tanh(b_ref[...] + r_ref[...] * agg)

    return pl.pallas_call(
        tc_kern,
        out_shape=jax.ShapeDtypeStruct((n, b), jnp.float32),
    )(partials, bias2, resp2)


@jax.jit
def kernel(x, states, edge_index, edge_weight, node_bias, node_response):
    n_nodes = node_bias.shape[0]
    n_edges = edge_weight.shape[0]
    in_size = x.shape[1]
    st = states.at[:, :in_size].set(x)
    st_t = st.T

    grain = NW * CHUNK * 2
    n_pad = (-n_edges) % grain
    pad_idx = jnp.arange(n_pad, dtype=jnp.int32) % n_nodes
    src = jnp.concatenate([edge_index[0], pad_idx])
    dst = jnp.concatenate([edge_index[1], pad_idx])
    w = jnp.pad(edge_weight, (0, n_pad))
    n_chunks = (n_edges + n_pad) // (NW * CHUNK)
    src3 = src.reshape(NW, n_chunks, CHUNK)
    dst3 = dst.reshape(NW, n_chunks, CHUNK)
    w3 = w.reshape(NW, n_chunks, CHUNK)

    zeros = jnp.zeros((1000, st_t.shape[1]), jnp.float32)
    partials = _sc_edge_scatter(st_t, src3, dst3, w3, zeros, n_nodes)
    act = _tc_finish(partials,
                     node_bias.reshape(n_nodes, 1),
                     node_response.reshape(n_nodes, 1))
    new_states = act.T
    new_states = new_states.at[:, :in_size].set(x)
    y = new_states[:, -OUT_SIZE:]
    return (y, new_states)


# speedup vs baseline: 1.4585x; 1.0595x over previous
import functools

import jax
import jax.numpy as jnp
from jax import lax
from jax.experimental import pallas as pl
from jax.experimental.pallas import tpu as pltpu
from jax.experimental.pallas import tpu_sc as plsc

NC = 2
NS = 16
NW = NC * NS
CHUNK = 128
OUT_SIZE = 64


def _sc_edge_scatter(st_t, src3, dst3, w3, zeros, n_nodes):
    b = st_t.shape[1]
    n_chunks = src3.shape[1]
    zrows = zeros.shape[0]
    nslices = n_nodes // zrows
    mesh = plsc.VectorSubcoreMesh(core_axis_name="c", subcore_axis_name="s")

    @functools.partial(
        pl.kernel,
        out_type=jax.ShapeDtypeStruct((NC, n_nodes, b), jnp.float32),
        mesh=mesh,
        scratch_types=[
            pltpu.VMEM((n_chunks, CHUNK), jnp.int32),
            pltpu.VMEM((n_chunks, CHUNK), jnp.int32),
            pltpu.VMEM((n_chunks, CHUNK), jnp.float32),
            pltpu.VMEM((CHUNK, b), jnp.float32),
            pltpu.VMEM((CHUNK, b), jnp.float32),
            pltpu.VMEM_SHARED((n_nodes, b), jnp.float32),
            pltpu.VMEM_SHARED((n_nodes, b), jnp.float32),
            pltpu.SemaphoreType.DMA,
            pltpu.SemaphoreType.DMA,
            pltpu.SemaphoreType.DMA,
            pltpu.SemaphoreType.DMA,
        ],
        compiler_params=pltpu.CompilerParams(use_tc_tiling_on_sc=False),
    )
    def sc_kern(st_hbm, src_hbm, dst_hbm, w_hbm, zeros_hbm, out_hbm,
                src_v, dst_v, w_v, rows_a, rows_b, acc_sh, st_sh,
                gsem_a, gsem_b, ssem_a, ssem_b):
        cid = lax.axis_index("c")
        sid = lax.axis_index("s")
        wid = sid * NC + cid

        pltpu.sync_copy(src_hbm.at[wid], src_v)
        pltpu.sync_copy(dst_hbm.at[wid], dst_v)
        pltpu.sync_copy(w_hbm.at[wid], w_v)

        @pl.when(sid < nslices)
        def _zero():
            sl = pl.ds(sid * zrows, zrows)
            pltpu.sync_copy(zeros_hbm, acc_sh.at[sl])
            pltpu.sync_copy(st_hbm.at[sl], st_sh.at[sl])
        plsc.subcore_barrier()

        def gather_start(c, buf, sem):
            pltpu.async_copy(st_sh.at[src_v.at[c]], buf, sem)

        def gather_wait(buf, sem):
            pltpu.make_async_copy(st_sh.at[src_v.at[0]], buf, sem).wait()

        def scale(c, buf):
            for g in range(CHUNK // 16):
                wvec = w_v[c, pl.ds(g * 16, 16)]
                for l in range(16):
                    wb = jnp.broadcast_to(wvec[l], (16,))
                    i = g * 16 + l
                    for j in range(b // 16):
                        sl = pl.ds(j * 16, 16)
                        buf[i, sl] = buf[i, sl] * wb

        def scatter_start(c, buf, sem):
            pltpu.async_copy(buf, acc_sh.at[dst_v.at[c]], sem, add=True)

        def scatter_wait(c, buf, sem):
            pltpu.make_async_copy(buf, acc_sh.at[dst_v.at[c]], sem).wait()

        gather_start(0, rows_a, gsem_a)

        def pair_body(k, _):
            c0 = 2 * k
            gather_wait(rows_a, gsem_a)
            gather_start(c0 + 1, rows_b, gsem_b)
            scale(c0, rows_a)
            scatter_start(c0, rows_a, ssem_a)
            gather_wait(rows_b, gsem_b)
            scale(c0 + 1, rows_b)
            scatter_start(c0 + 1, rows_b, ssem_b)
            scatter_wait(c0, rows_a, ssem_a)
            gather_start(c0 + 2, rows_a, gsem_a)
            scatter_wait(c0 + 1, rows_b, ssem_b)
            return 0
        lax.fori_loop(0, (n_chunks - 2) // 2, pair_body, 0)

        c1 = n_chunks - 2
        gather_wait(rows_a, gsem_a)
        gather_start(c1 + 1, rows_b, gsem_b)
        scale(c1, rows_a)
        scatter_start(c1, rows_a, ssem_a)
        gather_wait(rows_b, gsem_b)
        scale(c1 + 1, rows_b)
        scatter_start(c1 + 1, rows_b, ssem_b)
        scatter_wait(c1, rows_a, ssem_a)
        scatter_wait(c1 + 1, rows_b, ssem_b)

        plsc.subcore_barrier()

        @pl.when(sid < nslices)
        def _writeback():
            pltpu.sync_copy(acc_sh.at[pl.ds(sid * zrows, zrows)],
                            out_hbm.at[cid, pl.ds(sid * zrows, zrows)])

    return sc_kern(st_t, src3, dst3, w3, zeros)


def _tc_finish(partials, bias2, resp2):
    n, b = partials.shape[1], partials.shape[2]

    def tc_kern(p_ref, b_ref, r_ref, o_ref):
        agg = p_ref[0] + p_ref[1]
        o_ref[...] = jnp.tanh(b_ref[...] + r_ref[...] * agg)

    return pl.pallas_call(
        tc_kern,
        out_shape=jax.ShapeDtypeStruct((n, b), jnp.float32),
    )(partials, bias2, resp2)


@jax.jit
def kernel(x, states, edge_index, edge_weight, node_bias, node_response):
    n_nodes = node_bias.shape[0]
    n_edges = edge_weight.shape[0]
    in_size = x.shape[1]
    st = states.at[:, :in_size].set(x)
    st_t = st.T

    grain = NW * CHUNK * 2
    n_pad = (-n_edges) % grain
    pad_idx = jnp.arange(n_pad, dtype=jnp.int32) % n_nodes
    src = jnp.concatenate([edge_index[0], pad_idx])
    dst = jnp.concatenate([edge_index[1], pad_idx])
    w = jnp.pad(edge_weight, (0, n_pad))
    n_chunks = (n_edges + n_pad) // (NW * CHUNK)
    src3 = src.reshape(NW, n_chunks, CHUNK)
    dst3 = dst.reshape(NW, n_chunks, CHUNK)
    w3 = w.reshape(NW, n_chunks, CHUNK)

    zeros = jnp.zeros((1000, st_t.shape[1]), jnp.float32)
    partials = _sc_edge_scatter(st_t, src3, dst3, w3, zeros, n_nodes)
    act = _tc_finish(partials,
                     node_bias.reshape(n_nodes, 1),
                     node_response.reshape(n_nodes, 1))
    new_states = act.T
    new_states = new_states.at[:, :in_size].set(x)
    y = new_states[:, -OUT_SIZE:]
    return (y, new_states)


# speedup vs baseline: 1.5817x; 1.0845x over previous
import functools

import jax
import jax.numpy as jnp
from jax import lax
from jax.experimental import pallas as pl
from jax.experimental.pallas import tpu as pltpu
from jax.experimental.pallas import tpu_sc as plsc

NC = 2
NS = 16
NW = NC * NS
CHUNK = 80
OUT_SIZE = 64


def _sc_edge_scatter(st_t, src3, dst3, w3, zeros, n_nodes):
    b = st_t.shape[1]
    n_chunks = src3.shape[1]
    zrows = zeros.shape[0]
    nslices = n_nodes // zrows
    mesh = plsc.VectorSubcoreMesh(core_axis_name="c", subcore_axis_name="s")

    @functools.partial(
        pl.kernel,
        out_type=jax.ShapeDtypeStruct((NC, n_nodes, b), jnp.float32),
        mesh=mesh,
        scratch_types=[
            pltpu.VMEM((n_chunks, CHUNK), jnp.int32),
            pltpu.VMEM((n_chunks, CHUNK), jnp.int32),
            pltpu.VMEM((n_chunks, CHUNK), jnp.float32),
            pltpu.VMEM((CHUNK, b), jnp.float32),
            pltpu.VMEM((CHUNK, b), jnp.float32),
            pltpu.VMEM_SHARED((n_nodes, b), jnp.float32),
            pltpu.VMEM_SHARED((n_nodes, b), jnp.float32),
            pltpu.SemaphoreType.DMA,
            pltpu.SemaphoreType.DMA,
            pltpu.SemaphoreType.DMA,
            pltpu.SemaphoreType.DMA,
        ],
        compiler_params=pltpu.CompilerParams(use_tc_tiling_on_sc=False),
    )
    def sc_kern(st_hbm, src_hbm, dst_hbm, w_hbm, zeros_hbm, out_hbm,
                src_v, dst_v, w_v, rows_a, rows_b, acc_sh, st_sh,
                gsem_a, gsem_b, ssem_a, ssem_b):
        cid = lax.axis_index("c")
        sid = lax.axis_index("s")
        wid = sid * NC + cid

        pltpu.sync_copy(src_hbm.at[wid], src_v)
        pltpu.sync_copy(dst_hbm.at[wid], dst_v)
        pltpu.sync_copy(w_hbm.at[wid], w_v)

        @pl.when(sid < nslices)
        def _zero():
            sl = pl.ds(sid * zrows, zrows)
            pltpu.sync_copy(zeros_hbm, acc_sh.at[sl])
            pltpu.sync_copy(st_hbm.at[sl], st_sh.at[sl])
        plsc.subcore_barrier()

        def gather_start(c, buf, sem):
            pltpu.async_copy(st_sh.at[src_v.at[c]], buf, sem)

        def gather_wait(buf, sem):
            pltpu.make_async_copy(st_sh.at[src_v.at[0]], buf, sem).wait()

        def scatter_start(c, buf, sem):
            pltpu.async_copy(buf, acc_sh.at[dst_v.at[c]], sem, add=True)

        def scatter_wait(c, buf, sem):
            pltpu.make_async_copy(buf, acc_sh.at[dst_v.at[c]], sem).wait()

        def scale(c, buf):
            for g in range(CHUNK // 16):
                wvec = w_v[c, pl.ds(g * 16, 16)]
                for l in range(16):
                    wb = jnp.broadcast_to(wvec[l], (16,))
                    i = g * 16 + l
                    for j in range(b // 16):
                        sl = pl.ds(j * 16, 16)
                        buf[i, sl] = buf[i, sl] * wb

        gather_start(0, rows_a, gsem_a)

        def pair_body(k, _):
            c0 = 2 * k
            gather_wait(rows_a, gsem_a)
            gather_start(c0 + 1, rows_b, gsem_b)
            scale(c0, rows_a)
            scatter_start(c0, rows_a, ssem_a)
            gather_wait(rows_b, gsem_b)
            scale(c0 + 1, rows_b)
            scatter_start(c0 + 1, rows_b, ssem_b)
            scatter_wait(c0, rows_a, ssem_a)
            gather_start(c0 + 2, rows_a, gsem_a)
            scatter_wait(c0 + 1, rows_b, ssem_b)
            return 0
        lax.fori_loop(0, (n_chunks - 1) // 2, pair_body, 0)

        c1 = n_chunks - 1
        gather_wait(rows_a, gsem_a)
        scale(c1, rows_a)
        scatter_start(c1, rows_a, ssem_a)
        scatter_wait(c1, rows_a, ssem_a)

        plsc.subcore_barrier()

        @pl.when(sid < nslices)
        def _writeback():
            pltpu.sync_copy(acc_sh.at[pl.ds(sid * zrows, zrows)],
                            out_hbm.at[cid, pl.ds(sid * zrows, zrows)])

    return sc_kern(st_t, src3, dst3, w3, zeros)


def _tc_finish(partials, bias2, resp2, x):
    n, b = partials.shape[1], partials.shape[2]
    in_size = x.shape[1]

    def tc_kern(p_ref, b_ref, r_ref, x_ref, o_ref):
        agg = p_ref[0] + p_ref[1]
        act = jnp.tanh(b_ref[...] + r_ref[...] * agg)
        o_ref[...] = act.T
        o_ref[:, :in_size] = x_ref[...]

    return pl.pallas_call(
        tc_kern,
        out_shape=jax.ShapeDtypeStruct((b, n), jnp.float32),
    )(partials, bias2, resp2, x)


@jax.jit
def kernel(x, states, edge_index, edge_weight, node_bias, node_response):
    n_nodes = node_bias.shape[0]
    n_edges = edge_weight.shape[0]
    in_size = x.shape[1]
    st = states.at[:, :in_size].set(x)
    st_t = st.T

    n_chunks = n_edges // (NW * CHUNK)
    src3 = edge_index[0].reshape(NW, n_chunks, CHUNK)
    dst3 = edge_index[1].reshape(NW, n_chunks, CHUNK)
    w3 = edge_weight.reshape(NW, n_chunks, CHUNK)

    zeros = jnp.zeros((1000, st_t.shape[1]), jnp.float32)
    partials = _sc_edge_scatter(st_t, src3, dst3, w3, zeros, n_nodes)
    new_states = _tc_finish(partials,
                            node_bias.reshape(n_nodes, 1),
                            node_response.reshape(n_nodes, 1),
                            x)
    y = new_states[:, -OUT_SIZE:]
    return (y, new_states)
